# baseline (speedup 1.0000x reference)
import jax
import jax.numpy as jnp
from jax import lax
from jax.experimental import pallas as pl
from jax.experimental.pallas import tpu as pltpu
from jax.experimental.pallas import tpu_sc as plsc

NC = 2
NS = 16
NW = NC * NS
L = 16

ROWS = 128
COLS = 32768
SC_ROWS = 32
TC_ROWS = ROWS - SC_ROWS
ROWS_PER_W = SC_ROWS // NW
HALF = COLS // 2
U = 8
OUTER = HALF // (U * L)
BR = 8

_NEG_INF = float("-inf")
_BIG = 2**30


def _xlane(v, perm):
    return lax.gather(
        v,
        perm[:, None],
        lax.GatherDimensionNumbers(
            offset_dims=(), collapsed_slice_dims=(0,), start_index_map=(0,)
        ),
        slice_sizes=(1,),
        mode=lax.GatherScatterMode.PROMISE_IN_BOUNDS,
    )


def _merge(va, ia, vb, ib):
    take_a = (va > vb) | ((va == vb) & (ia < ib))
    return jnp.where(take_a, va, vb), jnp.where(take_a, ia, ib)


def _segment_scan(buf_ref):
    init = tuple(jnp.full((L,), _NEG_INF, jnp.float32) for _ in range(U)) + \
           tuple(jnp.zeros((L,), jnp.int32) for _ in range(U))

    def body(o, carry):
        vals = list(carry[:U])
        outs = list(carry[U:])
        o_vec = jnp.full((L,), o, jnp.int32)
        base = o * (U * L)
        for u in range(U):
            v = buf_ref[pl.ds(base + u * L, L)]
            take = v > vals[u]
            vals[u] = jnp.where(take, v, vals[u])
            outs[u] = jnp.where(take, o_vec, outs[u])
        return tuple(vals) + tuple(outs)

    res = lax.fori_loop(0, OUTER, body, init)
    return list(res[:U]), list(res[U:])


def _finalize_segment(vals, outs, seg_base, lane_iota):
    cand_v, cand_i = None, None
    for u in range(U):
        idx = outs[u] * (U * L) + (seg_base + u * L) + lane_iota
        if cand_v is None:
            cand_v, cand_i = vals[u], idx
        else:
            cand_v, cand_i = _merge(cand_v, cand_i, vals[u], idx)
    return cand_v, cand_i


def _sc_argmax_body(x_hbm, out_hbm, buf_ref, res_ref, sem0, sem1):
    wid = lax.axis_index("s") * NC + lax.axis_index("c")
    row0 = wid * ROWS_PER_W
    lane_iota = lax.iota(jnp.int32, L)
    sems = (sem0, sem1)

    def start(seg):
        r, h = divmod(seg, 2)
        b = seg % 2
        return pltpu.async_copy(
            x_hbm.at[row0 + r, pl.ds(h * HALF, HALF)], buf_ref.at[b], sems[b]
        )

    nseg = ROWS_PER_W * 2
    pending = start(0)
    res = jnp.zeros((L,), jnp.int32)

    row_v = row_i = None
    for seg in range(nseg):
        nxt = start(seg + 1) if seg + 1 < nseg else None
        pending.wait()
        pending = nxt
        r, h = divmod(seg, 2)
        vals, outs = _segment_scan(buf_ref.at[seg % 2])
        seg_v, seg_i = _finalize_segment(vals, outs, h * HALF, lane_iota)
        if h == 0:
            row_v, row_i = seg_v, seg_i
        else:
            row_v, row_i = _merge(row_v, row_i, seg_v, seg_i)
            for d in (8, 4, 2, 1):
                perm = lane_iota ^ d
                pv = _xlane(row_v, perm)
                pi = _xlane(row_i, perm)
                row_v, row_i = _merge(row_v, row_i, pv, pi)
            res = jnp.where(lane_iota == r, row_i, res)

    res_ref[...] = res
    pltpu.sync_copy(res_ref, out_hbm.at[wid])


def _sc_argmax(x_sc):
    mesh = plsc.VectorSubcoreMesh(
        core_axis_name="c", subcore_axis_name="s", num_cores=NC, num_subcores=NS
    )
    out = pl.kernel(
        _sc_argmax_body,
        out_type=jax.ShapeDtypeStruct((NW, L), jnp.int32),
        mesh=mesh,
        scratch_types=[
            pltpu.VMEM((2, HALF), jnp.float32),
            pltpu.VMEM((L,), jnp.int32),
            pltpu.SemaphoreType.DMA,
            pltpu.SemaphoreType.DMA,
        ],
    )(x_sc)
    return out[:, :ROWS_PER_W].reshape(SC_ROWS)


def _tc_body(x_ref, o_ref):
    v = x_ref[...]
    idx = lax.broadcasted_iota(jnp.int32, (BR, COLS), 1)
    m = jnp.max(v, axis=1, keepdims=True)
    cand = jnp.where(v == m, idx, _BIG)
    res = jnp.min(cand, axis=1)
    o_ref[...] = jnp.broadcast_to(res[:, None], (BR, 128))


def _tc_argmax(x_tc):
    out = pl.pallas_call(
        _tc_body,
        grid=(TC_ROWS // BR,),
        in_specs=[pl.BlockSpec((BR, COLS), lambda i: (i, 0))],
        out_specs=pl.BlockSpec((BR, 128), lambda i: (i, 0)),
        out_shape=jax.ShapeDtypeStruct((TC_ROWS, 128), jnp.int32),
    )(x_tc)
    return out[:, 0]


@jax.jit
def kernel(x):
    tc = _tc_argmax(x[:TC_ROWS])
    sc = _sc_argmax(x[TC_ROWS:])
    return jnp.concatenate([tc, sc])


# speedup vs baseline: 1.6868x; 1.6868x over previous
import jax
import jax.numpy as jnp
from jax import lax
from jax.experimental import pallas as pl
from jax.experimental.pallas import tpu as pltpu
from jax.experimental.pallas import tpu_sc as plsc

NC = 2
NS = 16
NW = NC * NS
L = 16

ROWS = 128
COLS = 32768
SC_ROWS = 32
TC_ROWS = ROWS - SC_ROWS
U = 8
OUTER = COLS // (U * L)
BR = 16

_NEG_INF = float("-inf")
_BIG = 2**30


def _xlane(v, perm):
    return lax.gather(
        v,
        perm[:, None],
        lax.GatherDimensionNumbers(
            offset_dims=(), collapsed_slice_dims=(0,), start_index_map=(0,)
        ),
        slice_sizes=(1,),
        mode=lax.GatherScatterMode.PROMISE_IN_BOUNDS,
    )


def _merge(va, ia, vb, ib):
    take_a = (va > vb) | ((va == vb) & (ia < ib))
    return jnp.where(take_a, va, vb), jnp.where(take_a, ia, ib)


def _sc_argmax_body(x_hbm, out_hbm, buf_ref, res_ref, sem0):
    wid = lax.axis_index("s") * NC + lax.axis_index("c")
    row = TC_ROWS + wid
    lane_iota = lax.iota(jnp.int32, L)

    pltpu.async_copy(x_hbm.at[row], buf_ref, sem0).wait()

    init = tuple(jnp.full((L,), _NEG_INF, jnp.float32) for _ in range(U)) + \
           tuple(jnp.zeros((L,), jnp.int32) for _ in range(U))

    def body(o, carry):
        vals = list(carry[:U])
        outs = list(carry[U:])
        o_vec = jnp.full((L,), o, jnp.int32)
        base = o * (U * L)
        for u in range(U):
            v = buf_ref[pl.ds(base + u * L, L)]
            take = v > vals[u]
            vals[u] = jnp.where(take, v, vals[u])
            outs[u] = jnp.where(take, o_vec, outs[u])
        return tuple(vals) + tuple(outs)

    acc = lax.fori_loop(0, OUTER, body, init)
    vals, outs = acc[:U], acc[U:]

    row_v = row_i = None
    for u in range(U):
        idx = outs[u] * (U * L) + (u * L) + lane_iota
        if row_v is None:
            row_v, row_i = vals[u], idx
        else:
            row_v, row_i = _merge(row_v, row_i, vals[u], idx)

    for d in (8, 4, 2, 1):
        perm = lane_iota ^ d
        pv = _xlane(row_v, perm)
        pi = _xlane(row_i, perm)
        row_v, row_i = _merge(row_v, row_i, pv, pi)

    res_ref[...] = row_i
    pltpu.sync_copy(res_ref, out_hbm.at[wid])


def _sc_argmax(x):
    mesh = plsc.VectorSubcoreMesh(
        core_axis_name="c", subcore_axis_name="s", num_cores=NC, num_subcores=NS
    )
    out = pl.kernel(
        _sc_argmax_body,
        out_type=jax.ShapeDtypeStruct((NW, L), jnp.int32),
        mesh=mesh,
        scratch_types=[
            pltpu.VMEM((COLS,), jnp.float32),
            pltpu.VMEM((L,), jnp.int32),
            pltpu.SemaphoreType.DMA,
        ],
    )(x)
    return out[:, 0]


def _tc_body(x_ref, o_ref):
    v = x_ref[...]
    idx = lax.broadcasted_iota(jnp.int32, (BR, COLS), 1)
    m = jnp.max(v, axis=1, keepdims=True)
    cand = jnp.where(v == m, idx, _BIG)
    res = jnp.min(cand, axis=1)
    o_ref[...] = jnp.broadcast_to(res[:, None], (BR, 128))


def _tc_argmax(x):
    out = pl.pallas_call(
        _tc_body,
        grid=(TC_ROWS // BR,),
        in_specs=[pl.BlockSpec((BR, COLS), lambda i: (i, 0))],
        out_specs=pl.BlockSpec((BR, 128), lambda i: (i, 0)),
        out_shape=jax.ShapeDtypeStruct((TC_ROWS, 128), jnp.int32),
    )(x)
    return out[:, 0]


@jax.jit
def kernel(x):
    tc = _tc_argmax(x)
    sc = _sc_argmax(x)
    return jnp.concatenate([tc, sc])


# speedup vs baseline: 5.6043x; 3.3225x over previous
import jax
import jax.numpy as jnp
from jax import lax
from jax.experimental import pallas as pl

ROWS = 128
COLS = 32768
BR = 64

_BIG = 2**30


HALF = COLS // 2


def _tc_body(x0_ref, x1_ref, o_ref):
    i = pl.program_id(0)
    v0 = x0_ref[...]
    v1 = x1_ref[...]
    idx = lax.broadcasted_iota(jnp.int32, (BR, HALF), 1)
    m0 = jnp.max(v0, axis=1, keepdims=True)
    m1 = jnp.max(v1, axis=1, keepdims=True)
    m = jnp.maximum(m0, m1)
    cand0 = jnp.where(v0 == m, idx, _BIG)
    cand1 = jnp.where(v1 == m, idx + HALF, _BIG)
    res = jnp.minimum(jnp.min(cand0, axis=1), jnp.min(cand1, axis=1))

    dup = jnp.concatenate([res, res]).reshape(1, ROWS)
    lane = lax.broadcasted_iota(jnp.int32, (1, ROWS), 1)
    keep = (lane // BR) == i
    prev = o_ref[...].reshape(1, ROWS)
    o_ref[...] = jnp.where(keep, dup, prev).reshape(ROWS)


@jax.jit
def kernel(x):
    return pl.pallas_call(
        _tc_body,
        grid=(ROWS // BR,),
        in_specs=[pl.BlockSpec((BR, HALF), lambda i: (i, 0)),
                  pl.BlockSpec((BR, HALF), lambda i: (i, 1))],
        out_specs=pl.BlockSpec((ROWS,), lambda i: (0,)),
        out_shape=jax.ShapeDtypeStruct((ROWS,), jnp.int32),
    )(x, x)
